# baseline (speedup 1.0000x reference)
import functools
import math

import jax
import jax.numpy as jnp
from jax import lax
from jax.experimental import pallas as pl
from jax.experimental.pallas import tpu as pltpu
from jax.experimental.pallas import tpu_sc as plsc

D_MODEL = 64
SCALE = math.sqrt(D_MODEL)
BLK = 128


def _emb_body(idx_hbm, lut_hbm, out_hbm, idx_v, rows0, rows1, tb0, tb1,
              gsem0, gsem1, osem0, osem1, *, nbatch, blocks_per_w):
    nc = plsc.get_sparse_core_info().num_cores
    wid = lax.axis_index("s") * nc + lax.axis_index("c")
    wbase = wid * blocks_per_w
    bpr = nbatch // BLK
    rows = (rows0, rows1)
    tbs = (tb0, tb1)
    gsems = (gsem0, gsem1)
    osems = (osem0, osem1)

    iota = lax.iota(jnp.int32, 16)
    rowi = [iota + 16 * j for j in range(BLK // 16)]

    def load_idx_and_fire(i, p):
        bi = wbase + i
        s = bi // bpr
        b0 = (bi % bpr) * BLK
        pltpu.sync_copy(idx_hbm.at[s].at[pl.ds(b0, BLK)], idx_v.at[p])
        pltpu.async_copy(lut_hbm.at[idx_v.at[p]], rows[p], gsems[p])

    def wait_gather(p):
        pltpu.make_async_copy(lut_hbm.at[idx_v.at[p]], rows[p],
                              gsems[p]).wait()

    def xpose_scale(p):
        @pl.loop(0, D_MODEL)
        def _(d):
            dcol = jnp.full((16,), d, dtype=jnp.int32)
            for j in range(BLK // 16):
                v = plsc.load_gather(rows[p], [rowi[j], dcol])
                tbs[p][d, pl.ds(16 * j, 16)] = v * SCALE

    def fire_store(i, p):
        bi = wbase + i
        s = bi // bpr
        b0 = (bi % bpr) * BLK
        pltpu.async_copy(tbs[p], out_hbm.at[s].at[:, pl.ds(b0, BLK)],
                         osems[p])

    def wait_store(p):
        pltpu.make_async_copy(tbs[p], out_hbm.at[0].at[:, pl.ds(0, BLK)],
                              osems[p]).wait()

    load_idx_and_fire(0, 0)
    load_idx_and_fire(1, 1)
    for i in range(2):
        wait_gather(i)
        xpose_scale(i)
        load_idx_and_fire(i + 2, i)
        fire_store(i, i)

    @pl.loop(1, blocks_per_w // 2 - 1)
    def _(go):
        for p in range(2):
            i = 2 * go + p
            wait_gather(p)
            wait_store(p)
            xpose_scale(p)
            load_idx_and_fire(i + 2, p)
            fire_store(i, p)

    for p in range(2):
        i = blocks_per_w - 2 + p
        wait_gather(p)
        wait_store(p)
        xpose_scale(p)
        fire_store(i, p)
    wait_store(0)
    wait_store(1)


def kernel(input_data, lut):
    s0, s1 = input_data.shape
    nbatch, nseq = s0, s1
    info = plsc.get_sparse_core_info()
    nw = info.num_cores * info.num_subcores
    nblocks = nbatch * nseq // BLK
    blocks_per_w = nblocks // nw
    assert nblocks % nw == 0 and blocks_per_w % 2 == 0

    idx_t = input_data.T.astype(jnp.int32)

    mesh = plsc.VectorSubcoreMesh(core_axis_name="c", subcore_axis_name="s")
    emb = functools.partial(
        pl.kernel,
        mesh=mesh,
        out_type=jax.ShapeDtypeStruct((nseq, D_MODEL, nbatch), jnp.float32),
        scratch_types=[
            pltpu.VMEM((2, BLK), jnp.int32),
            pltpu.VMEM((BLK, D_MODEL), jnp.float32),
            pltpu.VMEM((BLK, D_MODEL), jnp.float32),
            pltpu.VMEM((D_MODEL, BLK), jnp.float32),
            pltpu.VMEM((D_MODEL, BLK), jnp.float32),
            pltpu.SemaphoreType.DMA,
            pltpu.SemaphoreType.DMA,
            pltpu.SemaphoreType.DMA,
            pltpu.SemaphoreType.DMA,
        ],
        compiler_params=pltpu.CompilerParams(use_tc_tiling_on_sc=False,
                                             needs_layout_passes=False),
    )(functools.partial(_emb_body, nbatch=nbatch,
                        blocks_per_w=blocks_per_w))

    out = emb(idx_t, lut)
    return jnp.transpose(out, (2, 0, 1))


# speedup vs baseline: 1.8142x; 1.8142x over previous
import functools
import math

import jax
import jax.numpy as jnp
from jax import lax
from jax.experimental import pallas as pl
from jax.experimental.pallas import tpu as pltpu
from jax.experimental.pallas import tpu_sc as plsc

D_MODEL = 64
SCALE = math.sqrt(D_MODEL)
BLK = 128


def _emb_body(idx_hbm, lut_hbm, out_hbm, idx_v, idx2_v, rows0, rows1,
              tb0, tb1, gsem0, gsem1, osem0, osem1, *,
              nbatch, blocks_per_w):
    nc = plsc.get_sparse_core_info().num_cores
    wid = lax.axis_index("s") * nc + lax.axis_index("c")
    wbase = wid * blocks_per_w
    bpr = nbatch // BLK
    rows = (rows0, rows1)
    tbs = (tb0, tb1)
    gsems = (gsem0, gsem1)
    osems = (osem0, osem1)

    iota = lax.iota(jnp.int32, 16)
    rowi = [iota + 16 * j for j in range(BLK // 16)]

    def load_idx_and_fire(i, p):
        bi = wbase + i
        s = bi // bpr
        b0 = (bi % bpr) * BLK
        pltpu.sync_copy(idx_hbm.at[s].at[pl.ds(b0, BLK)], idx_v.at[p])
        for j in range(BLK // 16):
            v = idx_v[p, pl.ds(16 * j, 16)]
            idx2_v[p, pl.ds(16 * j, 16)] = lax.shift_right_logical(v, 1)
        pltpu.async_copy(lut_hbm.at[idx2_v.at[p]], rows[p], gsems[p])

    def wait_gather(p):
        pltpu.make_async_copy(lut_hbm.at[idx2_v.at[p]], rows[p],
                              gsems[p]).wait()

    def xpose_scale(p):
        paroff = [
            lax.shift_left(
                lax.bitwise_and(idx_v[p, pl.ds(16 * j, 16)], 1), 6)
            for j in range(BLK // 16)
        ]

        @pl.loop(0, D_MODEL)
        def _(d0):
            dvec = lax.bitwise_and(jnp.full((16,), d0, jnp.int32) + iota,
                                   D_MODEL - 1)
            for j in range(BLK // 16):
                c = dvec + paroff[j]
                v = plsc.load_gather(rows[p], [rowi[j], c])
                plsc.store_scatter(tbs[p], [dvec, rowi[j]], v * SCALE)

    def fire_store(i, p):
        bi = wbase + i
        s = bi // bpr
        b0 = (bi % bpr) * BLK
        pltpu.async_copy(tbs[p], out_hbm.at[s].at[:, pl.ds(b0, BLK)],
                         osems[p])

    def wait_store(p):
        pltpu.make_async_copy(tbs[p], out_hbm.at[0].at[:, pl.ds(0, BLK)],
                              osems[p]).wait()

    load_idx_and_fire(0, 0)
    load_idx_and_fire(1, 1)
    for i in range(2):
        wait_gather(i)
        xpose_scale(i)
        load_idx_and_fire(i + 2, i)
        fire_store(i, i)

    @pl.loop(1, blocks_per_w // 2 - 1)
    def _(go):
        for p in range(2):
            i = 2 * go + p
            wait_gather(p)
            wait_store(p)
            xpose_scale(p)
            load_idx_and_fire(i + 2, p)
            fire_store(i, p)

    for p in range(2):
        i = blocks_per_w - 2 + p
        wait_gather(p)
        wait_store(p)
        xpose_scale(p)
        fire_store(i, p)
    wait_store(0)
    wait_store(1)


def kernel(input_data, lut):
    nbatch, nseq = input_data.shape
    vocab = lut.shape[0]
    info = plsc.get_sparse_core_info()
    nw = info.num_cores * info.num_subcores
    nblocks = nbatch * nseq // BLK
    blocks_per_w = nblocks // nw
    assert nblocks % nw == 0 and blocks_per_w % 2 == 0

    idx_t = input_data.T.astype(jnp.int32)
    lut_p = lut.reshape(vocab // 2, 2 * D_MODEL)

    mesh = plsc.VectorSubcoreMesh(core_axis_name="c", subcore_axis_name="s")
    emb = functools.partial(
        pl.kernel,
        mesh=mesh,
        out_type=jax.ShapeDtypeStruct((nseq, D_MODEL, nbatch), jnp.float32),
        scratch_types=[
            pltpu.VMEM((2, BLK), jnp.int32),
            pltpu.VMEM((2, BLK), jnp.int32),
            pltpu.VMEM((BLK, 2 * D_MODEL), jnp.float32),
            pltpu.VMEM((BLK, 2 * D_MODEL), jnp.float32),
            pltpu.VMEM((D_MODEL, BLK), jnp.float32),
            pltpu.VMEM((D_MODEL, BLK), jnp.float32),
            pltpu.SemaphoreType.DMA,
            pltpu.SemaphoreType.DMA,
            pltpu.SemaphoreType.DMA,
            pltpu.SemaphoreType.DMA,
        ],
        compiler_params=pltpu.CompilerParams(use_tc_tiling_on_sc=True,
                                             needs_layout_passes=False),
    )(functools.partial(_emb_body, nbatch=nbatch,
                        blocks_per_w=blocks_per_w))

    out = emb(idx_t, lut_p)
    return jnp.transpose(out, (2, 0, 1))


# speedup vs baseline: 1.9569x; 1.0787x over previous
import functools
import math

import jax
import jax.numpy as jnp
from jax import lax
from jax.experimental import pallas as pl
from jax.experimental.pallas import tpu as pltpu
from jax.experimental.pallas import tpu_sc as plsc

D_MODEL = 64
SCALE = math.sqrt(D_MODEL)
BLK = 128
BPW = 4


def _emb_body(idx_hbm, lut_hbm, out_hbm, idx_v, idx2_v, rows0, rows1,
              tb0, tb1, gsem0, gsem1, osem0, osem1, *, nseq):
    nc = plsc.get_sparse_core_info().num_cores
    wid = lax.axis_index("s") * nc + lax.axis_index("c")
    nblocks = nseq * BPW
    rows = (rows0, rows1)
    tbs = (tb0, tb1)
    gsems = (gsem0, gsem1)
    osems = (osem0, osem1)

    iota = lax.iota(jnp.int32, 16)
    rowi = [iota + 16 * j for j in range(BLK // 16)]

    pltpu.sync_copy(idx_hbm.at[:, pl.ds(wid * BPW * BLK, BPW * BLK)], idx_v)

    def fire_gather(i, p):
        s = i // BPW
        q = i % BPW
        for j in range(BLK // 16):
            v = idx_v[s, pl.ds(q * BLK + 16 * j, 16)]
            idx2_v[p, pl.ds(16 * j, 16)] = lax.shift_right_logical(v, 1)
        pltpu.async_copy(lut_hbm.at[idx2_v.at[p]], rows[p], gsems[p])

    def wait_gather(p):
        pltpu.make_async_copy(lut_hbm.at[idx2_v.at[p]], rows[p],
                              gsems[p]).wait()

    def xpose_scale(i, p):
        s = i // BPW
        q = i % BPW
        paroff = [
            lax.shift_left(
                lax.bitwise_and(idx_v[s, pl.ds(q * BLK + 16 * j, 16)], 1),
                6)
            for j in range(BLK // 16)
        ]

        @pl.loop(0, D_MODEL)
        def _(d0):
            dvec = lax.bitwise_and(jnp.full((16,), d0, jnp.int32) + iota,
                                   D_MODEL - 1)
            for j in range(BLK // 16):
                c = dvec + paroff[j]
                v = plsc.load_gather(rows[p], [rowi[j], c])
                plsc.store_scatter(tbs[p], [dvec, rowi[j]], v * SCALE)

    def fire_store(i, p):
        s = i // BPW
        b0 = (wid * BPW + i % BPW) * BLK
        pltpu.async_copy(tbs[p], out_hbm.at[s].at[:, pl.ds(b0, BLK)],
                         osems[p])

    def wait_store(p):
        pltpu.make_async_copy(tbs[p], out_hbm.at[0].at[:, pl.ds(0, BLK)],
                              osems[p]).wait()

    fire_gather(0, 0)
    fire_gather(1, 1)
    for i in range(2):
        wait_gather(i)
        xpose_scale(i, i)
        fire_gather(i + 2, i)
        fire_store(i, i)

    @pl.loop(1, nblocks // 2 - 1)
    def _(go):
        for p in range(2):
            i = 2 * go + p
            wait_gather(p)
            wait_store(p)
            xpose_scale(i, p)
            fire_gather(i + 2, p)
            fire_store(i, p)

    for p in range(2):
        i = nblocks - 2 + p
        wait_gather(p)
        wait_store(p)
        xpose_scale(i, p)
        fire_store(i, p)
    wait_store(0)
    wait_store(1)


def kernel(input_data, lut):
    nbatch, nseq = input_data.shape
    vocab = lut.shape[0]
    info = plsc.get_sparse_core_info()
    nw = info.num_cores * info.num_subcores
    assert nbatch % (nw * BPW * BLK) == 0

    idx_t = input_data.T.astype(jnp.int32)
    lut_p = lut.reshape(vocab // 2, 2 * D_MODEL)

    mesh = plsc.VectorSubcoreMesh(core_axis_name="c", subcore_axis_name="s")
    emb = functools.partial(
        pl.kernel,
        mesh=mesh,
        out_type=jax.ShapeDtypeStruct((nseq, D_MODEL, nbatch), jnp.float32),
        scratch_types=[
            pltpu.VMEM((nseq, BPW * BLK), jnp.int32),
            pltpu.VMEM((2, BLK), jnp.int32),
            pltpu.VMEM((BLK, 2 * D_MODEL), jnp.float32),
            pltpu.VMEM((BLK, 2 * D_MODEL), jnp.float32),
            pltpu.VMEM((D_MODEL, BLK), jnp.float32),
            pltpu.VMEM((D_MODEL, BLK), jnp.float32),
            pltpu.SemaphoreType.DMA,
            pltpu.SemaphoreType.DMA,
            pltpu.SemaphoreType.DMA,
            pltpu.SemaphoreType.DMA,
        ],
        compiler_params=pltpu.CompilerParams(use_tc_tiling_on_sc=True,
                                             needs_layout_passes=False),
    )(functools.partial(_emb_body, nseq=nseq))

    out = emb(idx_t, lut_p)
    return jnp.transpose(out, (2, 0, 1))


# speedup vs baseline: 2.1303x; 1.0886x over previous
import functools
import math

import jax
import jax.numpy as jnp
from jax import lax
from jax.experimental import pallas as pl
from jax.experimental.pallas import tpu as pltpu
from jax.experimental.pallas import tpu_sc as plsc

D_MODEL = 64
SCALE = math.sqrt(D_MODEL)
BLK = 128
BPW = 4
VSTRIP = 128


def _nw():
    info = plsc.get_sparse_core_info()
    return info.num_cores * info.num_subcores


def _repack_body(lutt_hbm, out_hbm, in0, in1, ot0, ot1, tin, isem0, isem1,
                 osem0, osem1, *, vocab):
    nc = plsc.get_sparse_core_info().num_cores
    nw = _nw()
    wid = lax.axis_index("s") * nc + lax.axis_index("c")
    nstrips = vocab // VSTRIP
    nmain = nstrips // nw
    nextra = nstrips - nmain * nw
    ins = (in0, in1)
    ots = (ot0, ot1)
    isems = (isem0, isem1)
    osems = (osem0, osem1)

    iota = lax.iota(jnp.int32, 16)
    iotah = lax.shift_right_logical(iota, 1)
    par64 = lax.shift_left(lax.bitwise_and(iota, 1), 6)
    vvs = [iota + 16 * k for k in range(8)]
    rrs = [iotah + 8 * k for k in range(8)]

    def strip_of(n):
        return n * nw + wid

    def fire_in(n, p):
        pltpu.async_copy(lutt_hbm.at[:, pl.ds(strip_of(n) * VSTRIP, VSTRIP)],
                         ins[p], isems[p])

    def wait_in(p):
        pltpu.make_async_copy(lutt_hbm.at[:, pl.ds(0, VSTRIP)], ins[p],
                              isems[p]).wait()

    def xpose(src, dst, nk):
        @pl.loop(0, 16)
        def _(r):
            rot = lax.bitwise_and(iota + r, 15)
            for dq in range(4):
                dvec = rot + (16 * dq)
                cvec = dvec + par64
                for k in range(nk):
                    v = plsc.load_gather(src, [dvec, vvs[k]])
                    plsc.store_scatter(dst, [rrs[k], cvec], v)

    def fire_out(n, p):
        pltpu.async_copy(ots[p],
                         out_hbm.at[pl.ds(strip_of(n) * (VSTRIP // 2),
                                          VSTRIP // 2)],
                         osems[p])

    def wait_out(p):
        pltpu.make_async_copy(ots[p], out_hbm.at[pl.ds(0, VSTRIP // 2)],
                              osems[p]).wait()

    fire_in(0, 0)
    fire_in(1, 1)
    for n in range(2):
        wait_in(n)
        xpose(ins[n], ots[n], 8)
        fire_in(n + 2, n)
        fire_out(n, n)

    @pl.loop(1, nmain // 2 - 1)
    def _(go):
        for p in range(2):
            n = 2 * go + p
            wait_in(p)
            wait_out(p)
            xpose(ins[p], ots[p], 8)
            fire_in(n + 2, p)
            fire_out(n, p)

    for p in range(2):
        n = nmain - 2 + p
        wait_in(p)
        wait_out(p)
        xpose(ins[p], ots[p], 8)
        fire_out(n, p)
    wait_out(0)
    wait_out(1)

    @pl.when(wid < nextra)
    def _():
        t = nmain * nw + wid
        pltpu.sync_copy(lutt_hbm.at[:, pl.ds(t * VSTRIP, VSTRIP)], in0)
        xpose(in0, ot0, 8)
        pltpu.sync_copy(ot0, out_hbm.at[pl.ds(t * (VSTRIP // 2),
                                              VSTRIP // 2)])

    if vocab % VSTRIP != 0:
        @pl.when(wid == nextra)
        def _():
            v0 = nstrips * VSTRIP
            pltpu.sync_copy(lutt_hbm.at[:, pl.ds(v0, VSTRIP // 2)], tin)
            xpose(tin, ot1, 4)
            pltpu.sync_copy(ot1.at[pl.ds(0, VSTRIP // 4)],
                            out_hbm.at[pl.ds(v0 // 2, VSTRIP // 4)])


def _emb_body(idx_hbm, lut_hbm, out_hbm, idx_v, idx2_v, rows0, rows1,
              tb0, tb1, gsem0, gsem1, osem0, osem1, *, nseq):
    nc = plsc.get_sparse_core_info().num_cores
    wid = lax.axis_index("s") * nc + lax.axis_index("c")
    nblocks = nseq * BPW
    rows = (rows0, rows1)
    tbs = (tb0, tb1)
    gsems = (gsem0, gsem1)
    osems = (osem0, osem1)

    iota = lax.iota(jnp.int32, 16)
    rowi = [iota + 16 * j for j in range(BLK // 16)]

    pltpu.sync_copy(idx_hbm.at[:, pl.ds(wid * BPW * BLK, BPW * BLK)], idx_v)

    def fire_gather(i, p):
        s = i // BPW
        q = i % BPW
        for j in range(BLK // 16):
            v = idx_v[s, pl.ds(q * BLK + 16 * j, 16)]
            idx2_v[p, pl.ds(16 * j, 16)] = lax.shift_right_logical(v, 1)
        pltpu.async_copy(lut_hbm.at[idx2_v.at[p]], rows[p], gsems[p])

    def wait_gather(p):
        pltpu.make_async_copy(lut_hbm.at[idx2_v.at[p]], rows[p],
                              gsems[p]).wait()

    def xpose_scale(i, p):
        s = i // BPW
        q = i % BPW
        paroff = [
            lax.shift_left(
                lax.bitwise_and(idx_v[s, pl.ds(q * BLK + 16 * j, 16)], 1),
                6)
            for j in range(BLK // 16)
        ]

        @pl.loop(0, D_MODEL)
        def _(d0):
            dvec = lax.bitwise_and(jnp.full((16,), d0, jnp.int32) + iota,
                                   D_MODEL - 1)
            for j in range(BLK // 16):
                c = dvec + paroff[j]
                v = plsc.load_gather(rows[p], [rowi[j], c])
                plsc.store_scatter(tbs[p], [dvec, rowi[j]], v * SCALE)

    def fire_store(i, p):
        s = i // BPW
        b0 = (wid * BPW + i % BPW) * BLK
        pltpu.async_copy(tbs[p], out_hbm.at[s].at[:, pl.ds(b0, BLK)],
                         osems[p])

    def wait_store(p):
        pltpu.make_async_copy(tbs[p], out_hbm.at[0].at[:, pl.ds(0, BLK)],
                              osems[p]).wait()

    fire_gather(0, 0)
    fire_gather(1, 1)
    for i in range(2):
        wait_gather(i)
        xpose_scale(i, i)
        fire_gather(i + 2, i)
        fire_store(i, i)

    @pl.loop(1, nblocks // 2 - 1)
    def _(go):
        for p in range(2):
            i = 2 * go + p
            wait_gather(p)
            wait_store(p)
            xpose_scale(i, p)
            fire_gather(i + 2, p)
            fire_store(i, p)

    for p in range(2):
        i = nblocks - 2 + p
        wait_gather(p)
        wait_store(p)
        xpose_scale(i, p)
        fire_store(i, p)
    wait_store(0)
    wait_store(1)


def kernel(input_data, lut):
    nbatch, nseq = input_data.shape
    vocab = lut.shape[0]
    nw = _nw()
    assert nbatch % (nw * BPW * BLK) == 0 and vocab % 2 == 0

    idx_t = input_data.T.astype(jnp.int32)
    lut_t = lut.T

    mesh = plsc.VectorSubcoreMesh(core_axis_name="c", subcore_axis_name="s")
    cparams = pltpu.CompilerParams(use_tc_tiling_on_sc=True,
                                   needs_layout_passes=False)

    repack = functools.partial(
        pl.kernel,
        mesh=mesh,
        out_type=jax.ShapeDtypeStruct((vocab // 2, 2 * D_MODEL),
                                      jnp.float32),
        scratch_types=[
            pltpu.VMEM((D_MODEL, VSTRIP), jnp.float32),
            pltpu.VMEM((D_MODEL, VSTRIP), jnp.float32),
            pltpu.VMEM((VSTRIP // 2, 2 * D_MODEL), jnp.float32),
            pltpu.VMEM((VSTRIP // 2, 2 * D_MODEL), jnp.float32),
            pltpu.VMEM((D_MODEL, VSTRIP // 2), jnp.float32),
            pltpu.SemaphoreType.DMA,
            pltpu.SemaphoreType.DMA,
            pltpu.SemaphoreType.DMA,
            pltpu.SemaphoreType.DMA,
        ],
        compiler_params=cparams,
    )(functools.partial(_repack_body, vocab=vocab))

    emb = functools.partial(
        pl.kernel,
        mesh=mesh,
        out_type=jax.ShapeDtypeStruct((nseq, D_MODEL, nbatch), jnp.float32),
        scratch_types=[
            pltpu.VMEM((nseq, BPW * BLK), jnp.int32),
            pltpu.VMEM((2, BLK), jnp.int32),
            pltpu.VMEM((BLK, 2 * D_MODEL), jnp.float32),
            pltpu.VMEM((BLK, 2 * D_MODEL), jnp.float32),
            pltpu.VMEM((D_MODEL, BLK), jnp.float32),
            pltpu.VMEM((D_MODEL, BLK), jnp.float32),
            pltpu.SemaphoreType.DMA,
            pltpu.SemaphoreType.DMA,
            pltpu.SemaphoreType.DMA,
            pltpu.SemaphoreType.DMA,
        ],
        compiler_params=cparams,
    )(functools.partial(_emb_body, nseq=nseq))

    lut_packed = repack(lut_t)
    out = emb(idx_t, lut_packed)
    return jnp.transpose(out, (2, 0, 1))
